# Initial kernel scaffold; baseline (speedup 1.0000x reference)
import jax
import jax.numpy as jnp
from jax.experimental import pallas as pl


def kernel(node_attr, edge_attr, edge_index, W, b):
    raise NotImplementedError("write your pallas kernel here")


# speedup vs baseline: 5.8997x; 5.8997x over previous
import functools

import jax
import jax.numpy as jnp
from jax import lax
from jax.experimental import pallas as pl
from jax.experimental.pallas import tpu as pltpu
from jax.experimental.pallas import tpu_sc as plsc

N_NODES = 10000
N_EDGES = 320000
D_FEAT = 128
OUT_CH = 64

NC = 2
NS = 16
NW = NC * NS

SUB = 128

A_CHUNK = 256
A_ROWS = A_CHUNK // SUB
A_CHUNKS = N_EDGES // A_CHUNK
A_BASE = A_CHUNKS // NW
A_EXTRA = A_CHUNKS - A_BASE * NW

G_CHUNK = 512
G_ROWS = G_CHUNK // SUB
G_TOTAL = 2 * N_EDGES
G_CHUNKS = G_TOTAL // G_CHUNK
G_BASE = G_CHUNKS // NW
G_EXTRA = G_CHUNKS - G_BASE * NW

ZROWS = 1000
CNT_PER_TILE = 1000

_mesh = plsc.VectorSubcoreMesh(core_axis_name="c", subcore_axis_name="s")


@functools.partial(
    pl.kernel,
    out_type=(
        jax.ShapeDtypeStruct((2 * N_NODES, D_FEAT), jnp.float32),
        jax.ShapeDtypeStruct((2 * N_NODES,), jnp.float32),
    ),
    mesh=_mesh,
    scratch_types=(
        pltpu.VMEM_SHARED((N_NODES, D_FEAT), jnp.float32),
        pltpu.VMEM_SHARED((N_NODES,), jnp.float32),
        pltpu.VMEM((A_CHUNK, D_FEAT), jnp.float32),
        pltpu.VMEM((A_ROWS, SUB), jnp.int32),
        pltpu.VMEM((SUB,), jnp.float32),
        pltpu.VMEM((CNT_PER_TILE,), jnp.float32),
    ),
)
def _scatter_phase(edge_hbm, idx_hbm, z2d_hbm, psums_hbm, pcnt_hbm,
                   sums_sh, cnt_sh, ebuf, ibuf, ones, cbuf):
    c = lax.axis_index("c")
    s = lax.axis_index("s")
    w = c * NS + s

    for i in range(SUB // 16):
        ones[pl.ds(i * 16, 16)] = jnp.ones((16,), jnp.float32)

    for i in range(CNT_PER_TILE // 16):
        cbuf[pl.ds(i * 16, 16)] = jnp.zeros((16,), jnp.float32)

    @pl.when(s < NS - 6)
    def _():
        pltpu.sync_copy(z2d_hbm, sums_sh.at[pl.ds(s * ZROWS, ZROWS)])
        pltpu.sync_copy(cbuf, cnt_sh.at[pl.ds(s * CNT_PER_TILE, CNT_PER_TILE)])

    plsc.subcore_barrier()

    nch = A_BASE + jnp.where(w < A_EXTRA, 1, 0)

    def body(k, _):
        cid = w + NW * k
        base = cid * A_CHUNK
        pltpu.sync_copy(edge_hbm.at[pl.ds(base, A_CHUNK)], ebuf)
        pltpu.sync_copy(idx_hbm.at[cid], ibuf)
        for j in range(A_ROWS):
            pltpu.sync_copy(ebuf.at[pl.ds(j * SUB, SUB)],
                            sums_sh.at[ibuf.at[j]], add=True)
            pltpu.sync_copy(ones, cnt_sh.at[ibuf.at[j]], add=True)
        return _

    lax.fori_loop(0, nch, body, None)

    plsc.subcore_barrier()

    @pl.when(s < NS - 6)
    def _():
        pltpu.sync_copy(sums_sh.at[pl.ds(s * ZROWS, ZROWS)],
                        psums_hbm.at[pl.ds(c * N_NODES + s * ZROWS, ZROWS)])
        pltpu.sync_copy(cnt_sh.at[pl.ds(s * CNT_PER_TILE, CNT_PER_TILE)], cbuf)
        pltpu.sync_copy(cbuf,
                        pcnt_hbm.at[pl.ds(c * N_NODES + s * CNT_PER_TILE, CNT_PER_TILE)])


@functools.partial(
    pl.kernel,
    out_type=jax.ShapeDtypeStruct((G_TOTAL, OUT_CH), jnp.float32),
    mesh=_mesh,
    scratch_types=(
        pltpu.VMEM((G_CHUNK, OUT_CH), jnp.float32),
        pltpu.VMEM((G_ROWS, SUB), jnp.int32),
    ),
    compiler_params=pltpu.CompilerParams(use_tc_tiling_on_sc=False),
)
def _gather_phase(emb_hbm, idx_hbm, out_hbm, gbuf, ibuf):
    c = lax.axis_index("c")
    s = lax.axis_index("s")
    w = c * NS + s

    nch = G_BASE + jnp.where(w < G_EXTRA, 1, 0)

    def body(k, _):
        cid = w + NW * k
        pltpu.sync_copy(idx_hbm.at[cid], ibuf)
        for j in range(G_ROWS):
            pltpu.sync_copy(emb_hbm.at[ibuf.at[j]], gbuf.at[pl.ds(j * SUB, SUB)])
        pltpu.sync_copy(gbuf, out_hbm.at[pl.ds(cid * G_CHUNK, G_CHUNK)])
        return _

    lax.fori_loop(0, nch, body, None)


def _update_body(node_ref, ps_ref, pc_ref, w_ref, b_ref, o_ref):
    sums = ps_ref[0] + ps_ref[1]
    cnt = jnp.maximum(pc_ref[0] + pc_ref[1], 1.0)
    mean = sums / cnt
    h = (node_ref[...] + mean) * 0.5
    z = lax.dot_general(h, w_ref[...], (((1,), (1,)), ((), ())),
                        preferred_element_type=jnp.float32)
    o_ref[...] = jax.nn.sigmoid(z + b_ref[...])


_N_BLK = 2000


def _node_update(node_attr, psums, pcnt, W, b):
    grid = N_NODES // _N_BLK
    return pl.pallas_call(
        _update_body,
        grid=(grid,),
        in_specs=[
            pl.BlockSpec((_N_BLK, D_FEAT), lambda i: (i, 0)),
            pl.BlockSpec((2, _N_BLK, D_FEAT), lambda i: (0, i, 0)),
            pl.BlockSpec((2, _N_BLK, 1), lambda i: (0, i, 0)),
            pl.BlockSpec((OUT_CH, D_FEAT), lambda i: (0, 0)),
            pl.BlockSpec((1, OUT_CH), lambda i: (0, 0)),
        ],
        out_specs=pl.BlockSpec((_N_BLK, OUT_CH), lambda i: (i, 0)),
        out_shape=jax.ShapeDtypeStruct((N_NODES, OUT_CH), jnp.float32),
    )(node_attr, psums, pcnt, W, b)


def kernel(node_attr, edge_attr, edge_index, W, b):
    ei = edge_index.astype(jnp.int32)
    src2d = ei[0].reshape(A_CHUNKS, A_ROWS, SUB)
    idx_inter = jnp.stack([ei[0], ei[1]], axis=1).reshape(G_CHUNKS, G_ROWS, SUB)
    z2d = jnp.zeros((ZROWS, D_FEAT), jnp.float32)

    psums, pcnt = _scatter_phase(edge_attr, src2d, z2d)
    emb = _node_update(node_attr,
                       psums.reshape(2, N_NODES, D_FEAT),
                       pcnt.reshape(2, N_NODES, 1),
                       W, b.reshape(1, OUT_CH))
    out64 = _gather_phase(emb, idx_inter)
    return out64.reshape(N_EDGES, D_FEAT)


# speedup vs baseline: 7.4204x; 1.2578x over previous
import functools

import jax
import jax.numpy as jnp
from jax import lax
from jax.experimental import pallas as pl
from jax.experimental.pallas import tpu as pltpu
from jax.experimental.pallas import tpu_sc as plsc

N_NODES = 10000
N_EDGES = 320000
D_FEAT = 128
OUT_CH = 64

NC = 2
NS = 16
NW = NC * NS

SUB = 128

A_CHUNK = 128
A_CHUNKS = N_EDGES // A_CHUNK
A_BASE = A_CHUNKS // NW
A_EXTRA = A_CHUNKS - A_BASE * NW

G_CHUNK = 512
G_ROWS = G_CHUNK // SUB
G_TOTAL = 2 * N_EDGES
G_CHUNKS = G_TOTAL // G_CHUNK
G_BASE = G_CHUNKS // NW
G_EXTRA = G_CHUNKS - G_BASE * NW

ZROWS = 1000
CNT_PER_TILE = 1000

_mesh = plsc.VectorSubcoreMesh(core_axis_name="c", subcore_axis_name="s")


@functools.partial(
    pl.kernel,
    out_type=(
        jax.ShapeDtypeStruct((2 * N_NODES, D_FEAT), jnp.float32),
        jax.ShapeDtypeStruct((2 * N_NODES,), jnp.float32),
    ),
    mesh=_mesh,
    scratch_types=(
        pltpu.VMEM_SHARED((N_NODES, D_FEAT), jnp.float32),
        pltpu.VMEM_SHARED((N_NODES,), jnp.float32),
        pltpu.VMEM((2, A_CHUNK, D_FEAT), jnp.float32),
        pltpu.VMEM((2, 1, SUB), jnp.int32),
        pltpu.VMEM((SUB,), jnp.float32),
        pltpu.VMEM((CNT_PER_TILE,), jnp.float32),
        pltpu.SemaphoreType.DMA,
        pltpu.SemaphoreType.DMA,
    ),
)
def _scatter_phase(edge_hbm, idx_hbm, z2d_hbm, psums_hbm, pcnt_hbm,
                   sums_sh, cnt_sh, ebuf, ibuf, ones, cbuf, sem0, sem1):
    c = lax.axis_index("c")
    s = lax.axis_index("s")
    w = c * NS + s
    sems = (sem0, sem1)

    for i in range(SUB // 16):
        ones[pl.ds(i * 16, 16)] = jnp.ones((16,), jnp.float32)

    for i in range(CNT_PER_TILE // 16):
        cbuf[pl.ds(i * 16, 16)] = jnp.zeros((16,), jnp.float32)

    @pl.when(s < NS - 6)
    def _():
        pltpu.sync_copy(z2d_hbm, sums_sh.at[pl.ds(s * ZROWS, ZROWS)])
        pltpu.sync_copy(cbuf, cnt_sh.at[pl.ds(s * CNT_PER_TILE, CNT_PER_TILE)])

    plsc.subcore_barrier()

    nch = A_BASE + jnp.where(w < A_EXTRA, 1, 0)

    def start_load(k, b):
        cid = w + NW * k
        pltpu.async_copy(edge_hbm.at[pl.ds(cid * A_CHUNK, A_CHUNK)],
                         ebuf.at[b], sems[b])
        pltpu.async_copy(idx_hbm.at[cid], ibuf.at[b], sems[b])

    def wait_load(k, b):
        cid = w + NW * k
        pltpu.make_async_copy(edge_hbm.at[pl.ds(cid * A_CHUNK, A_CHUNK)],
                              ebuf.at[b], sems[b]).wait()
        pltpu.make_async_copy(idx_hbm.at[cid], ibuf.at[b], sems[b]).wait()

    def scatter(b):
        pltpu.sync_copy(ebuf.at[b], sums_sh.at[ibuf.at[b, 0]], add=True)
        pltpu.sync_copy(ones, cnt_sh.at[ibuf.at[b, 0]], add=True)

    start_load(0, 0)

    def body(m, _):
        k0 = 2 * m
        k1 = k0 + 1

        @pl.when(k1 < nch)
        def _():
            start_load(k1, 1)

        wait_load(k0, 0)
        scatter(0)

        @pl.when(k1 < nch)
        def _():
            @pl.when(k1 + 1 < nch)
            def _():
                start_load(k1 + 1, 0)

            wait_load(k1, 1)
            scatter(1)

        return _

    lax.fori_loop(0, (nch + 1) // 2, body, None)

    plsc.subcore_barrier()

    @pl.when(s < NS - 6)
    def _():
        pltpu.sync_copy(sums_sh.at[pl.ds(s * ZROWS, ZROWS)],
                        psums_hbm.at[pl.ds(c * N_NODES + s * ZROWS, ZROWS)])
        pltpu.sync_copy(cnt_sh.at[pl.ds(s * CNT_PER_TILE, CNT_PER_TILE)], cbuf)
        pltpu.sync_copy(cbuf,
                        pcnt_hbm.at[pl.ds(c * N_NODES + s * CNT_PER_TILE, CNT_PER_TILE)])


@functools.partial(
    pl.kernel,
    out_type=jax.ShapeDtypeStruct((G_TOTAL, OUT_CH), jnp.float32),
    mesh=_mesh,
    scratch_types=(
        pltpu.VMEM((2, G_CHUNK, OUT_CH), jnp.float32),
        pltpu.VMEM((2, G_ROWS, SUB), jnp.int32),
        pltpu.SemaphoreType.DMA,
        pltpu.SemaphoreType.DMA,
        pltpu.SemaphoreType.DMA,
        pltpu.SemaphoreType.DMA,
        pltpu.SemaphoreType.DMA,
    ),
    compiler_params=pltpu.CompilerParams(use_tc_tiling_on_sc=False),
)
def _gather_phase(emb_hbm, idx_hbm, out_hbm, gbuf, ibuf,
                  semi0, semi1, semg, semo0, semo1):
    c = lax.axis_index("c")
    s = lax.axis_index("s")
    w = c * NS + s
    semi = (semi0, semi1)
    semo = (semo0, semo1)

    nch = G_BASE + jnp.where(w < G_EXTRA, 1, 0)

    def start_idx(k, b):
        cid = w + NW * k
        pltpu.async_copy(idx_hbm.at[cid], ibuf.at[b], semi[b])

    def wait_idx(k, b):
        cid = w + NW * k
        pltpu.make_async_copy(idx_hbm.at[cid], ibuf.at[b], semi[b]).wait()

    def start_write(k, b):
        cid = w + NW * k
        pltpu.async_copy(gbuf.at[b], out_hbm.at[pl.ds(cid * G_CHUNK, G_CHUNK)],
                         semo[b])

    def wait_write(k, b):
        cid = w + NW * k
        pltpu.make_async_copy(gbuf.at[b],
                              out_hbm.at[pl.ds(cid * G_CHUNK, G_CHUNK)],
                              semo[b]).wait()

    def gathers(b):
        for j in range(G_ROWS):
            pltpu.async_copy(emb_hbm.at[ibuf.at[b, j]],
                             gbuf.at[b, pl.ds(j * SUB, SUB)], semg)
        for j in range(G_ROWS):
            pltpu.make_async_copy(emb_hbm.at[ibuf.at[b, j]],
                                  gbuf.at[b, pl.ds(j * SUB, SUB)], semg).wait()

    def chunk_body(k, m, b):
        wait_idx(k, b)

        @pl.when(k + 1 < nch)
        def _():
            start_idx(k + 1, 1 - b)

        @pl.when(m > 0)
        def _():
            wait_write(k - 2, b)

        gathers(b)
        start_write(k, b)

    start_idx(0, 0)

    def body(m, _):
        k0 = 2 * m
        chunk_body(k0, m, 0)

        @pl.when(k0 + 1 < nch)
        def _():
            chunk_body(k0 + 1, m, 1)

        return _

    lax.fori_loop(0, (nch + 1) // 2, body, None)

    wait_write(nch - 1 - ((nch - 1) % 2), 0)
    wait_write(nch - 1 - (nch % 2), 1)


def _update_body(node_ref, ps_ref, pc_ref, w_ref, b_ref, o_ref):
    sums = ps_ref[0] + ps_ref[1]
    cnt = jnp.maximum(pc_ref[0] + pc_ref[1], 1.0)
    mean = sums / cnt
    h = (node_ref[...] + mean) * 0.5
    z = lax.dot_general(h, w_ref[...], (((1,), (1,)), ((), ())),
                        preferred_element_type=jnp.float32)
    o_ref[...] = jax.nn.sigmoid(z + b_ref[...])


_N_BLK = 2000


def _node_update(node_attr, psums, pcnt, W, b):
    grid = N_NODES // _N_BLK
    return pl.pallas_call(
        _update_body,
        grid=(grid,),
        in_specs=[
            pl.BlockSpec((_N_BLK, D_FEAT), lambda i: (i, 0)),
            pl.BlockSpec((2, _N_BLK, D_FEAT), lambda i: (0, i, 0)),
            pl.BlockSpec((2, _N_BLK, 1), lambda i: (0, i, 0)),
            pl.BlockSpec((OUT_CH, D_FEAT), lambda i: (0, 0)),
            pl.BlockSpec((1, OUT_CH), lambda i: (0, 0)),
        ],
        out_specs=pl.BlockSpec((_N_BLK, OUT_CH), lambda i: (i, 0)),
        out_shape=jax.ShapeDtypeStruct((N_NODES, OUT_CH), jnp.float32),
    )(node_attr, psums, pcnt, W, b)


def kernel(node_attr, edge_attr, edge_index, W, b):
    ei = edge_index.astype(jnp.int32)
    src2d = ei[0].reshape(A_CHUNKS, 1, SUB)
    idx_inter = jnp.stack([ei[0], ei[1]], axis=1).reshape(G_CHUNKS, G_ROWS, SUB)
    z2d = jnp.zeros((ZROWS, D_FEAT), jnp.float32)

    psums, pcnt = _scatter_phase(edge_attr, src2d, z2d)
    emb = _node_update(node_attr,
                       psums.reshape(2, N_NODES, D_FEAT),
                       pcnt.reshape(2, N_NODES, 1),
                       W, b.reshape(1, OUT_CH))
    out64 = _gather_phase(emb, idx_inter)
    return out64.reshape(N_EDGES, D_FEAT)


# speedup vs baseline: 10.2842x; 1.3859x over previous
import functools

import jax
import jax.numpy as jnp
from jax import lax
from jax.experimental import pallas as pl
from jax.experimental.pallas import tpu as pltpu
from jax.experimental.pallas import tpu_sc as plsc

N_NODES = 10000
N_EDGES = 320000
D_FEAT = 128
OUT_CH = 64

NC = 2
NS = 16
NW = NC * NS

SUB = 128

A_CHUNK = 128
A_CHUNKS = N_EDGES // A_CHUNK
A_BASE = A_CHUNKS // NW
A_EXTRA = A_CHUNKS - A_BASE * NW

G_CHUNK = 256
G_ROWS = G_CHUNK // SUB
G_CHUNKS = N_EDGES // G_CHUNK
G_BASE = G_CHUNKS // NW
G_EXTRA = G_CHUNKS - G_BASE * NW

ZROWS = 1000
CNT_PER_TILE = 1000

_mesh = plsc.VectorSubcoreMesh(core_axis_name="c", subcore_axis_name="s")


@functools.partial(
    pl.kernel,
    out_type=(
        jax.ShapeDtypeStruct((2, N_NODES, D_FEAT), jnp.float32),
        jax.ShapeDtypeStruct((2 * N_NODES,), jnp.float32),
    ),
    mesh=_mesh,
    scratch_types=(
        pltpu.VMEM_SHARED((N_NODES, D_FEAT), jnp.float32),
        pltpu.VMEM_SHARED((N_NODES,), jnp.float32),
        pltpu.VMEM((2, A_CHUNK, D_FEAT), jnp.float32),
        pltpu.VMEM((2, 1, SUB), jnp.int32),
        pltpu.VMEM((SUB,), jnp.float32),
        pltpu.VMEM((CNT_PER_TILE,), jnp.float32),
        pltpu.SemaphoreType.DMA,
        pltpu.SemaphoreType.DMA,
    ),
)
def _scatter_phase(edge_hbm, idx_hbm, z2d_hbm, psums_hbm, pcnt_hbm,
                   sums_sh, cnt_sh, ebuf, ibuf, ones, cbuf, sem0, sem1):
    c = lax.axis_index("c")
    s = lax.axis_index("s")
    w = c * NS + s
    sems = (sem0, sem1)

    for i in range(SUB // 16):
        ones[pl.ds(i * 16, 16)] = jnp.ones((16,), jnp.float32)

    for i in range(CNT_PER_TILE // 16):
        cbuf[pl.ds(i * 16, 16)] = jnp.zeros((16,), jnp.float32)

    @pl.when(s < NS - 6)
    def _():
        pltpu.sync_copy(z2d_hbm, sums_sh.at[pl.ds(s * ZROWS, ZROWS)])
        pltpu.sync_copy(cbuf, cnt_sh.at[pl.ds(s * CNT_PER_TILE, CNT_PER_TILE)])

    plsc.subcore_barrier()

    nch = A_BASE + jnp.where(w < A_EXTRA, 1, 0)

    def start_load(k, b):
        cid = w + NW * k
        pltpu.async_copy(edge_hbm.at[pl.ds(cid * A_CHUNK, A_CHUNK)],
                         ebuf.at[b], sems[b])
        pltpu.async_copy(idx_hbm.at[cid], ibuf.at[b], sems[b])

    def wait_load(k, b):
        cid = w + NW * k
        pltpu.make_async_copy(edge_hbm.at[pl.ds(cid * A_CHUNK, A_CHUNK)],
                              ebuf.at[b], sems[b]).wait()
        pltpu.make_async_copy(idx_hbm.at[cid], ibuf.at[b], sems[b]).wait()

    def scatter(b):
        pltpu.sync_copy(ebuf.at[b], sums_sh.at[ibuf.at[b, 0]], add=True)
        pltpu.sync_copy(ones, cnt_sh.at[ibuf.at[b, 0]], add=True)

    start_load(0, 0)

    def body(m, _):
        k0 = 2 * m
        k1 = k0 + 1

        @pl.when(k1 < nch)
        def _():
            start_load(k1, 1)

        wait_load(k0, 0)
        scatter(0)

        @pl.when(k1 < nch)
        def _():
            @pl.when(k1 + 1 < nch)
            def _():
                start_load(k1 + 1, 0)

            wait_load(k1, 1)
            scatter(1)

        return _

    lax.fori_loop(0, (nch + 1) // 2, body, None)

    plsc.subcore_barrier()

    @pl.when(s < NS - 6)
    def _():
        pltpu.sync_copy(sums_sh.at[pl.ds(s * ZROWS, ZROWS)],
                        psums_hbm.at[c, pl.ds(s * ZROWS, ZROWS)])
        pltpu.sync_copy(cnt_sh.at[pl.ds(s * CNT_PER_TILE, CNT_PER_TILE)], cbuf)
        pltpu.sync_copy(cbuf,
                        pcnt_hbm.at[pl.ds(c * N_NODES + s * CNT_PER_TILE, CNT_PER_TILE)])


@functools.partial(
    pl.kernel,
    out_type=jax.ShapeDtypeStruct((N_EDGES, D_FEAT), jnp.float32),
    mesh=_mesh,
    scratch_types=(
        pltpu.VMEM((2, G_CHUNK, OUT_CH), jnp.float32),
        pltpu.VMEM((2, G_CHUNK, OUT_CH), jnp.float32),
        pltpu.VMEM((2, G_ROWS, SUB), jnp.int32),
        pltpu.VMEM((2, G_ROWS, SUB), jnp.int32),
        pltpu.SemaphoreType.DMA,
        pltpu.SemaphoreType.DMA,
        pltpu.SemaphoreType.DMA,
        pltpu.SemaphoreType.DMA,
        pltpu.SemaphoreType.DMA,
    ),
    compiler_params=pltpu.CompilerParams(use_tc_tiling_on_sc=False),
)
def _gather_phase(emb_hbm, sidx_hbm, didx_hbm, out_hbm, abuf, bbuf, sibuf, dibuf,
                  semi0, semi1, semg, semo0, semo1):
    c = lax.axis_index("c")
    s = lax.axis_index("s")
    w = c * NS + s
    semi = (semi0, semi1)
    semo = (semo0, semo1)

    nch = G_BASE + jnp.where(w < G_EXTRA, 1, 0)

    def start_idx(k, b):
        cid = w + NW * k
        pltpu.async_copy(sidx_hbm.at[cid], sibuf.at[b], semi[b])
        pltpu.async_copy(didx_hbm.at[cid], dibuf.at[b], semi[b])

    def wait_idx(k, b):
        cid = w + NW * k
        pltpu.make_async_copy(sidx_hbm.at[cid], sibuf.at[b], semi[b]).wait()
        pltpu.make_async_copy(didx_hbm.at[cid], dibuf.at[b], semi[b]).wait()

    def start_write(k, b):
        cid = w + NW * k
        pltpu.async_copy(abuf.at[b],
                         out_hbm.at[pl.ds(cid * G_CHUNK, G_CHUNK), pl.ds(0, OUT_CH)],
                         semo[b])
        pltpu.async_copy(bbuf.at[b],
                         out_hbm.at[pl.ds(cid * G_CHUNK, G_CHUNK), pl.ds(OUT_CH, OUT_CH)],
                         semo[b])

    def wait_write(k, b):
        cid = w + NW * k
        pltpu.make_async_copy(abuf.at[b],
                              out_hbm.at[pl.ds(cid * G_CHUNK, G_CHUNK), pl.ds(0, OUT_CH)],
                              semo[b]).wait()
        pltpu.make_async_copy(bbuf.at[b],
                              out_hbm.at[pl.ds(cid * G_CHUNK, G_CHUNK), pl.ds(OUT_CH, OUT_CH)],
                              semo[b]).wait()

    def gathers(b):
        for j in range(G_ROWS):
            pltpu.async_copy(emb_hbm.at[sibuf.at[b, j]],
                             abuf.at[b, pl.ds(j * SUB, SUB)], semg)
            pltpu.async_copy(emb_hbm.at[dibuf.at[b, j]],
                             bbuf.at[b, pl.ds(j * SUB, SUB)], semg)
        for j in range(G_ROWS):
            pltpu.make_async_copy(emb_hbm.at[sibuf.at[b, j]],
                                  abuf.at[b, pl.ds(j * SUB, SUB)], semg).wait()
            pltpu.make_async_copy(emb_hbm.at[dibuf.at[b, j]],
                                  bbuf.at[b, pl.ds(j * SUB, SUB)], semg).wait()

    def chunk_body(k, m, b):
        wait_idx(k, b)

        @pl.when(k + 1 < nch)
        def _():
            start_idx(k + 1, 1 - b)

        @pl.when(m > 0)
        def _():
            wait_write(k - 2, b)

        gathers(b)
        start_write(k, b)

    start_idx(0, 0)

    def body(m, _):
        k0 = 2 * m
        chunk_body(k0, m, 0)

        @pl.when(k0 + 1 < nch)
        def _():
            chunk_body(k0 + 1, m, 1)

        return _

    lax.fori_loop(0, (nch + 1) // 2, body, None)

    wait_write(nch - 1 - ((nch - 1) % 2), 0)
    wait_write(nch - 1 - (nch % 2), 1)


def _update_body(node_ref, ps_ref, pc_ref, w_ref, b_ref, o_ref):
    sums = ps_ref[0] + ps_ref[1]
    cnt = jnp.maximum(pc_ref[0] + pc_ref[1], 1.0)
    mean = sums / cnt
    h = (node_ref[...] + mean) * 0.5
    z = lax.dot_general(h, w_ref[...], (((1,), (1,)), ((), ())),
                        preferred_element_type=jnp.float32)
    o_ref[...] = jax.nn.sigmoid(z + b_ref[...])


_N_BLK = 2000


def _node_update(node_attr, psums, pcnt, W, b):
    grid = N_NODES // _N_BLK
    return pl.pallas_call(
        _update_body,
        grid=(grid,),
        in_specs=[
            pl.BlockSpec((_N_BLK, D_FEAT), lambda i: (i, 0)),
            pl.BlockSpec((2, _N_BLK, D_FEAT), lambda i: (0, i, 0)),
            pl.BlockSpec((2, _N_BLK, 1), lambda i: (0, i, 0)),
            pl.BlockSpec((OUT_CH, D_FEAT), lambda i: (0, 0)),
            pl.BlockSpec((1, OUT_CH), lambda i: (0, 0)),
        ],
        out_specs=pl.BlockSpec((_N_BLK, OUT_CH), lambda i: (i, 0)),
        out_shape=jax.ShapeDtypeStruct((N_NODES, OUT_CH), jnp.float32),
    )(node_attr, psums, pcnt, W, b)


def kernel(node_attr, edge_attr, edge_index, W, b):
    ei = edge_index.astype(jnp.int32)
    src2d = ei[0].reshape(A_CHUNKS, 1, SUB)
    src3 = ei[0].reshape(G_CHUNKS, G_ROWS, SUB)
    dst3 = ei[1].reshape(G_CHUNKS, G_ROWS, SUB)
    z2d = jnp.zeros((ZROWS, D_FEAT), jnp.float32)

    psums, pcnt = _scatter_phase(edge_attr, src2d, z2d)
    emb = _node_update(node_attr, psums, pcnt.reshape(2, N_NODES, 1),
                       W, b.reshape(1, OUT_CH))
    return _gather_phase(emb, src3, dst3)


# speedup vs baseline: 12.8911x; 1.2535x over previous
import functools

import jax
import jax.numpy as jnp
from jax import lax
from jax.experimental import pallas as pl
from jax.experimental.pallas import tpu as pltpu
from jax.experimental.pallas import tpu_sc as plsc

N_NODES = 10000
N_EDGES = 320000
D_FEAT = 128
OUT_CH = 64

NC = 2
NS = 16
NW = NC * NS

SUB = 128

A_CHUNK = 128
A_CHUNKS = N_EDGES // A_CHUNK
A_BASE = A_CHUNKS // NW
A_EXTRA = A_CHUNKS - A_BASE * NW

G_CHUNK = 256
G_ROWS = G_CHUNK // SUB
G_CHUNKS = N_EDGES // G_CHUNK
G_BASE = G_CHUNKS // NW
G_EXTRA = G_CHUNKS - G_BASE * NW

ZROWS = 1000
CNT_PER_TILE = 1000

_mesh = plsc.VectorSubcoreMesh(core_axis_name="c", subcore_axis_name="s")


@functools.partial(
    pl.kernel,
    out_type=(
        jax.ShapeDtypeStruct((2, N_NODES, D_FEAT), jnp.float32),
        jax.ShapeDtypeStruct((2 * N_NODES,), jnp.float32),
    ),
    mesh=_mesh,
    scratch_types=(
        pltpu.VMEM_SHARED((N_NODES, D_FEAT), jnp.float32),
        pltpu.VMEM_SHARED((N_NODES,), jnp.float32),
        pltpu.VMEM((2, A_CHUNK, D_FEAT), jnp.float32),
        pltpu.VMEM((2, 1, SUB), jnp.int32),
        pltpu.VMEM((SUB,), jnp.float32),
        pltpu.VMEM((CNT_PER_TILE,), jnp.float32),
        pltpu.SemaphoreType.DMA,
        pltpu.SemaphoreType.DMA,
    ),
)
def _scatter_phase(edge_hbm, idx_hbm, z2d_hbm, psums_hbm, pcnt_hbm,
                   sums_sh, cnt_sh, ebuf, ibuf, ones, cbuf, sem0, sem1):
    c = lax.axis_index("c")
    s = lax.axis_index("s")
    w = c * NS + s
    sems = (sem0, sem1)

    for i in range(SUB // 16):
        ones[pl.ds(i * 16, 16)] = jnp.ones((16,), jnp.float32)

    for i in range(CNT_PER_TILE // 16):
        cbuf[pl.ds(i * 16, 16)] = jnp.zeros((16,), jnp.float32)

    @pl.when(s < NS - 6)
    def _():
        pltpu.sync_copy(z2d_hbm, sums_sh.at[pl.ds(s * ZROWS, ZROWS)])
        pltpu.sync_copy(cbuf, cnt_sh.at[pl.ds(s * CNT_PER_TILE, CNT_PER_TILE)])

    plsc.subcore_barrier()

    nch = A_BASE + jnp.where(w < A_EXTRA, 1, 0)

    def start_load(k, b):
        cid = w + NW * k
        pltpu.async_copy(edge_hbm.at[pl.ds(cid * A_CHUNK, A_CHUNK)],
                         ebuf.at[b], sems[b])
        pltpu.async_copy(idx_hbm.at[cid], ibuf.at[b], sems[b])

    def wait_load(k, b):
        cid = w + NW * k
        pltpu.make_async_copy(edge_hbm.at[pl.ds(cid * A_CHUNK, A_CHUNK)],
                              ebuf.at[b], sems[b]).wait()
        pltpu.make_async_copy(idx_hbm.at[cid], ibuf.at[b], sems[b]).wait()

    def scatter(b):
        pltpu.sync_copy(ebuf.at[b], sums_sh.at[ibuf.at[b, 0]], add=True)
        pltpu.sync_copy(ones, cnt_sh.at[ibuf.at[b, 0]], add=True)

    start_load(0, 0)

    def body(m, _):
        k0 = 2 * m
        k1 = k0 + 1

        @pl.when(k1 < nch)
        def _():
            start_load(k1, 1)

        wait_load(k0, 0)
        scatter(0)

        @pl.when(k1 < nch)
        def _():
            @pl.when(k1 + 1 < nch)
            def _():
                start_load(k1 + 1, 0)

            wait_load(k1, 1)
            scatter(1)

        return _

    lax.fori_loop(0, (nch + 1) // 2, body, None)

    plsc.subcore_barrier()

    @pl.when(s < NS - 6)
    def _():
        pltpu.sync_copy(sums_sh.at[pl.ds(s * ZROWS, ZROWS)],
                        psums_hbm.at[c, pl.ds(s * ZROWS, ZROWS)])
        pltpu.sync_copy(cnt_sh.at[pl.ds(s * CNT_PER_TILE, CNT_PER_TILE)], cbuf)
        pltpu.sync_copy(cbuf,
                        pcnt_hbm.at[pl.ds(c * N_NODES + s * CNT_PER_TILE, CNT_PER_TILE)])


EROWS = 1000


@functools.partial(
    pl.kernel,
    out_type=jax.ShapeDtypeStruct((N_EDGES, D_FEAT), jnp.float32),
    mesh=_mesh,
    scratch_types=(
        pltpu.VMEM_SHARED((N_NODES, OUT_CH), jnp.float32),
        pltpu.VMEM((2, G_CHUNK, OUT_CH), jnp.float32),
        pltpu.VMEM((2, G_CHUNK, OUT_CH), jnp.float32),
        pltpu.VMEM((2, G_ROWS, SUB), jnp.int32),
        pltpu.VMEM((2, G_ROWS, SUB), jnp.int32),
        pltpu.SemaphoreType.DMA,
        pltpu.SemaphoreType.DMA,
        pltpu.SemaphoreType.DMA,
        pltpu.SemaphoreType.DMA,
        pltpu.SemaphoreType.DMA,
    ),
    compiler_params=pltpu.CompilerParams(use_tc_tiling_on_sc=False),
)
def _gather_phase(emb_hbm, sidx_hbm, didx_hbm, out_hbm, emb_sh, abuf, bbuf,
                  sibuf, dibuf, semi0, semi1, semg, semo0, semo1):
    c = lax.axis_index("c")
    s = lax.axis_index("s")
    w = c * NS + s
    semi = (semi0, semi1)
    semo = (semo0, semo1)

    @pl.when(s < NS - 6)
    def _():
        pltpu.sync_copy(emb_hbm.at[pl.ds(s * EROWS, EROWS)],
                        emb_sh.at[pl.ds(s * EROWS, EROWS)])

    nch = G_BASE + jnp.where(w < G_EXTRA, 1, 0)

    def start_idx(k, b):
        cid = w + NW * k
        pltpu.async_copy(sidx_hbm.at[cid], sibuf.at[b], semi[b])
        pltpu.async_copy(didx_hbm.at[cid], dibuf.at[b], semi[b])

    def wait_idx(k, b):
        cid = w + NW * k
        pltpu.make_async_copy(sidx_hbm.at[cid], sibuf.at[b], semi[b]).wait()
        pltpu.make_async_copy(didx_hbm.at[cid], dibuf.at[b], semi[b]).wait()

    def start_write(k, b):
        cid = w + NW * k
        pltpu.async_copy(abuf.at[b],
                         out_hbm.at[pl.ds(cid * G_CHUNK, G_CHUNK), pl.ds(0, OUT_CH)],
                         semo[b])
        pltpu.async_copy(bbuf.at[b],
                         out_hbm.at[pl.ds(cid * G_CHUNK, G_CHUNK), pl.ds(OUT_CH, OUT_CH)],
                         semo[b])

    def wait_write(k, b):
        cid = w + NW * k
        pltpu.make_async_copy(abuf.at[b],
                              out_hbm.at[pl.ds(cid * G_CHUNK, G_CHUNK), pl.ds(0, OUT_CH)],
                              semo[b]).wait()
        pltpu.make_async_copy(bbuf.at[b],
                              out_hbm.at[pl.ds(cid * G_CHUNK, G_CHUNK), pl.ds(OUT_CH, OUT_CH)],
                              semo[b]).wait()

    def gathers(b):
        for j in range(G_ROWS):
            pltpu.async_copy(emb_sh.at[sibuf.at[b, j]],
                             abuf.at[b, pl.ds(j * SUB, SUB)], semg)
            pltpu.async_copy(emb_sh.at[dibuf.at[b, j]],
                             bbuf.at[b, pl.ds(j * SUB, SUB)], semg)
        for j in range(G_ROWS):
            pltpu.make_async_copy(emb_sh.at[sibuf.at[b, j]],
                                  abuf.at[b, pl.ds(j * SUB, SUB)], semg).wait()
            pltpu.make_async_copy(emb_sh.at[dibuf.at[b, j]],
                                  bbuf.at[b, pl.ds(j * SUB, SUB)], semg).wait()

    def chunk_body(k, m, b):
        wait_idx(k, b)

        @pl.when(k + 1 < nch)
        def _():
            start_idx(k + 1, 1 - b)

        @pl.when(m > 0)
        def _():
            wait_write(k - 2, b)

        gathers(b)
        start_write(k, b)

    start_idx(0, 0)
    plsc.subcore_barrier()

    def body(m, _):
        k0 = 2 * m
        chunk_body(k0, m, 0)

        @pl.when(k0 + 1 < nch)
        def _():
            chunk_body(k0 + 1, m, 1)

        return _

    lax.fori_loop(0, (nch + 1) // 2, body, None)

    wait_write(nch - 1 - ((nch - 1) % 2), 0)
    wait_write(nch - 1 - (nch % 2), 1)


def _update_body(node_ref, ps_ref, pc_ref, w_ref, b_ref, o_ref):
    sums = ps_ref[0] + ps_ref[1]
    cnt = jnp.maximum(pc_ref[0] + pc_ref[1], 1.0)
    mean = sums / cnt
    h = (node_ref[...] + mean) * 0.5
    z = lax.dot_general(h, w_ref[...], (((1,), (1,)), ((), ())),
                        preferred_element_type=jnp.float32)
    o_ref[...] = jax.nn.sigmoid(z + b_ref[...])


_N_BLK = 2000


def _node_update(node_attr, psums, pcnt, W, b):
    grid = N_NODES // _N_BLK
    return pl.pallas_call(
        _update_body,
        grid=(grid,),
        in_specs=[
            pl.BlockSpec((_N_BLK, D_FEAT), lambda i: (i, 0)),
            pl.BlockSpec((2, _N_BLK, D_FEAT), lambda i: (0, i, 0)),
            pl.BlockSpec((2, _N_BLK, 1), lambda i: (0, i, 0)),
            pl.BlockSpec((OUT_CH, D_FEAT), lambda i: (0, 0)),
            pl.BlockSpec((1, OUT_CH), lambda i: (0, 0)),
        ],
        out_specs=pl.BlockSpec((_N_BLK, OUT_CH), lambda i: (i, 0)),
        out_shape=jax.ShapeDtypeStruct((N_NODES, OUT_CH), jnp.float32),
    )(node_attr, psums, pcnt, W, b)


def kernel(node_attr, edge_attr, edge_index, W, b):
    ei = edge_index.astype(jnp.int32)
    src2d = ei[0].reshape(A_CHUNKS, 1, SUB)
    src3 = ei[0].reshape(G_CHUNKS, G_ROWS, SUB)
    dst3 = ei[1].reshape(G_CHUNKS, G_ROWS, SUB)
    z2d = jnp.zeros((ZROWS, D_FEAT), jnp.float32)

    psums, pcnt = _scatter_phase(edge_attr, src2d, z2d)
    emb = _node_update(node_attr, psums, pcnt.reshape(2, N_NODES, 1),
                       W, b.reshape(1, OUT_CH))
    return _gather_phase(emb, src3, dst3)
